# Initial kernel scaffold; baseline (speedup 1.0000x reference)
import jax
import jax.numpy as jnp
from jax.experimental import pallas as pl


def kernel(x, table):
    raise NotImplementedError("write your pallas kernel here")


# speedup vs baseline: 1.6074x; 1.6074x over previous
import functools

import jax
import jax.numpy as jnp
from jax import lax
from jax.experimental import pallas as pl
from jax.experimental.pallas import tpu as pltpu
from jax.experimental.pallas import tpu_sc as plsc

NC = 2
NS = 16
NW = NC * NS


@functools.partial(jax.jit, static_argnums=(2, 3))
def _sc_gather(idx, table, n_chunks, chunk):
    n_total = idx.shape[0] * idx.shape[1] * idx.shape[2]
    d = table.shape[1]
    mesh = plsc.VectorSubcoreMesh(core_axis_name="c", subcore_axis_name="s")

    @functools.partial(
        pl.kernel,
        mesh=mesh,
        out_type=jax.ShapeDtypeStruct((n_total, d), jnp.float32),
        scratch_types=[
            pltpu.VMEM((n_chunks, chunk), jnp.int32),
            pltpu.VMEM((chunk, d), jnp.float32),
            pltpu.SemaphoreType.DMA,
        ],
    )
    def k(table_hbm, idx_hbm, out_hbm, idx_v, rows_v, gsem):
        wid = lax.axis_index("s") * NC + lax.axis_index("c")
        base = wid * n_chunks * chunk
        pltpu.sync_copy(idx_hbm.at[wid], idx_v)

        def step(j, carry):
            pltpu.async_copy(table_hbm.at[idx_v.at[j]], rows_v, gsem).wait()
            pltpu.sync_copy(rows_v, out_hbm.at[pl.ds(base + j * chunk, chunk)])
            return carry

        lax.fori_loop(0, n_chunks, step, 0)

    return k(table, idx)


def kernel(x, table):
    b, s = x.shape
    n_total = b * s
    d = table.shape[1]
    chunk = 80
    n_chunks = n_total // (NW * chunk)
    idx = x.reshape(NW, n_chunks, chunk)
    out = _sc_gather(idx, table, n_chunks, chunk)
    return out.reshape(b, s, d)


# speedup vs baseline: 1.8144x; 1.1288x over previous
import functools

import jax
import jax.numpy as jnp
from jax import lax
from jax.experimental import pallas as pl
from jax.experimental.pallas import tpu as pltpu
from jax.experimental.pallas import tpu_sc as plsc

NC = 2
NS = 16
NW = NC * NS


@functools.partial(jax.jit, static_argnums=(2, 3))
def _sc_gather(idx, table, n_chunks, chunk):
    n_total = idx.shape[0] * idx.shape[1] * idx.shape[2]
    d = table.shape[1]
    mesh = plsc.VectorSubcoreMesh(core_axis_name="c", subcore_axis_name="s")

    @functools.partial(
        pl.kernel,
        mesh=mesh,
        out_type=jax.ShapeDtypeStruct((n_total, d), jnp.float32),
        scratch_types=[
            pltpu.VMEM((n_chunks, chunk), jnp.int32),
            pltpu.VMEM((chunk, d), jnp.float32),
            pltpu.VMEM((chunk, d), jnp.float32),
            pltpu.SemaphoreType.DMA,
            pltpu.SemaphoreType.DMA,
            pltpu.SemaphoreType.DMA,
            pltpu.SemaphoreType.DMA,
        ],
    )
    def k(table_hbm, idx_hbm, out_hbm, idx_v, rows0, rows1, g0, g1, s0, s1):
        wid = lax.axis_index("s") * NC + lax.axis_index("c")
        base = wid * n_chunks * chunk
        pltpu.sync_copy(idx_hbm.at[wid], idx_v)

        def out_at(j):
            return out_hbm.at[pl.ds(base + j * chunk, chunk)]

        def body(jj, carry):
            j0 = 2 * jj
            j1 = j0 + 1
            pl.when(jj > 0)(
                lambda: pltpu.make_async_copy(rows0, out_at(j0), s0).wait())
            pltpu.async_copy(table_hbm.at[idx_v.at[j0]], rows0, g0)
            pl.when(jj > 0)(
                lambda: pltpu.make_async_copy(rows1, out_at(j1), s1).wait())
            pltpu.async_copy(table_hbm.at[idx_v.at[j1]], rows1, g1)
            pltpu.make_async_copy(table_hbm.at[idx_v.at[j0]], rows0, g0).wait()
            pltpu.async_copy(rows0, out_at(j0), s0)
            pltpu.make_async_copy(table_hbm.at[idx_v.at[j1]], rows1, g1).wait()
            pltpu.async_copy(rows1, out_at(j1), s1)
            return carry

        last = n_chunks // 2 - 1
        lax.fori_loop(0, n_chunks // 2, body, 0)
        pltpu.make_async_copy(rows0, out_at(2 * last), s0).wait()
        pltpu.make_async_copy(rows1, out_at(2 * last + 1), s1).wait()

    return k(table, idx)


def kernel(x, table):
    b, s = x.shape
    n_total = b * s
    d = table.shape[1]
    chunk = 80
    n_chunks = n_total // (NW * chunk)
    idx = x.reshape(NW, n_chunks, chunk)
    out = _sc_gather(idx, table, n_chunks, chunk)
    return out.reshape(b, s, d)


# speedup vs baseline: 1.8304x; 1.0088x over previous
import functools

import jax
import jax.numpy as jnp
from jax import lax
from jax.experimental import pallas as pl
from jax.experimental.pallas import tpu as pltpu
from jax.experimental.pallas import tpu_sc as plsc

NC = 2
NS = 16
NW = NC * NS
NBUF = 4


@functools.partial(jax.jit, static_argnums=(2, 3))
def _sc_gather(idx, table, n_chunks, chunk):
    n_total = idx.shape[0] * idx.shape[1] * idx.shape[2]
    d = table.shape[1]
    mesh = plsc.VectorSubcoreMesh(core_axis_name="c", subcore_axis_name="s")

    @functools.partial(
        pl.kernel,
        mesh=mesh,
        out_type=jax.ShapeDtypeStruct((n_total, d), jnp.float32),
        scratch_types=[
            pltpu.VMEM((n_chunks, chunk), jnp.int32),
            pltpu.VMEM((NBUF, chunk, d), jnp.float32),
            [pltpu.SemaphoreType.DMA] * NBUF,
            [pltpu.SemaphoreType.DMA] * NBUF,
        ],
    )
    def k(table_hbm, idx_hbm, out_hbm, idx_v, rows, gsems, ssems):
        wid = lax.axis_index("s") * NC + lax.axis_index("c")
        base = wid * n_chunks * chunk
        pltpu.sync_copy(idx_hbm.at[wid], idx_v)

        def out_at(j):
            return out_hbm.at[pl.ds(base + j * chunk, chunk)]

        def body(jj, carry):
            for b in range(NBUF):
                j = NBUF * jj + b
                pl.when(jj > 0)(
                    lambda b=b, j=j: pltpu.make_async_copy(
                        rows.at[b], out_at(j), ssems[b]).wait())
                pltpu.async_copy(table_hbm.at[idx_v.at[j]], rows.at[b], gsems[b])
            for b in range(NBUF):
                j = NBUF * jj + b
                pltpu.make_async_copy(
                    table_hbm.at[idx_v.at[j]], rows.at[b], gsems[b]).wait()
                pltpu.async_copy(rows.at[b], out_at(j), ssems[b])
            return carry

        lax.fori_loop(0, n_chunks // NBUF, body, 0)
        last = n_chunks - NBUF
        for b in range(NBUF):
            pltpu.make_async_copy(rows.at[b], out_at(last + b), ssems[b]).wait()

    return k(table, idx)


def kernel(x, table):
    b, s = x.shape
    n_total = b * s
    d = table.shape[1]
    chunk = 40
    n_chunks = n_total // (NW * chunk)
    idx = x.reshape(NW, n_chunks, chunk)
    out = _sc_gather(idx, table, n_chunks, chunk)
    return out.reshape(b, s, d)


# speedup vs baseline: 1.8480x; 1.0096x over previous
import functools

import jax
import jax.numpy as jnp
from jax import lax
from jax.experimental import pallas as pl
from jax.experimental.pallas import tpu as pltpu
from jax.experimental.pallas import tpu_sc as plsc

NC = 2
NS = 16
NW = NC * NS
NBUF = 3


@functools.partial(jax.jit, static_argnums=(2, 3))
def _sc_gather(idx, table, n_chunks, chunk):
    n_total = idx.shape[0] * idx.shape[1] * idx.shape[2]
    d = table.shape[1]
    mesh = plsc.VectorSubcoreMesh(core_axis_name="c", subcore_axis_name="s")

    @functools.partial(
        pl.kernel,
        mesh=mesh,
        out_type=jax.ShapeDtypeStruct((n_total, d), jnp.float32),
        scratch_types=[
            pltpu.VMEM((n_chunks, chunk), jnp.int32),
            pltpu.VMEM((NBUF, chunk, d), jnp.float32),
            [pltpu.SemaphoreType.DMA] * NBUF,
            [pltpu.SemaphoreType.DMA] * NBUF,
        ],
    )
    def k(table_hbm, idx_hbm, out_hbm, idx_v, rows, gsems, ssems):
        wid = lax.axis_index("s") * NC + lax.axis_index("c")
        base = wid * n_chunks * chunk
        pltpu.sync_copy(idx_hbm.at[wid], idx_v)

        def out_at(j):
            return out_hbm.at[pl.ds(base + j * chunk, chunk)]

        def body(jj, carry):
            for b in range(NBUF):
                j = NBUF * jj + b
                pl.when(jj > 0)(
                    lambda b=b, j=j: pltpu.make_async_copy(
                        rows.at[b], out_at(j), ssems[b]).wait())
                pltpu.async_copy(table_hbm.at[idx_v.at[j]], rows.at[b], gsems[b])
            for b in range(NBUF):
                j = NBUF * jj + b
                pltpu.make_async_copy(
                    table_hbm.at[idx_v.at[j]], rows.at[b], gsems[b]).wait()
                pltpu.async_copy(rows.at[b], out_at(j), ssems[b])
            return carry

        lax.fori_loop(0, n_chunks // NBUF, body, 0)
        last = n_chunks - NBUF
        for b in range(NBUF):
            pltpu.make_async_copy(rows.at[b], out_at(last + b), ssems[b]).wait()

    return k(table, idx)


def kernel(x, table):
    b, s = x.shape
    n_total = b * s
    d = table.shape[1]
    chunk = 64
    n_chunks = n_total // (NW * chunk)
    idx = x.reshape(NW, n_chunks, chunk)
    out = _sc_gather(idx, table, n_chunks, chunk)
    return out.reshape(b, s, d)
